# baseline (speedup 1.0000x reference)
import functools

import jax
import jax.numpy as jnp
from jax import lax
from jax.experimental import pallas as pl
from jax.experimental.pallas import tpu as pltpu
from jax.experimental.pallas import tpu_sc as plsc

B = 1024
L = 200
N = B * L
D_WORD = 300
D_PAD = 384
D_CHAR = 64
V_WORD = 100000
V_CHAR = 128
W = 16
HIDDEN = D_WORD + D_CHAR

NC = 2
NS = 16
NW = NC * NS
B_PER_W = N // NW
CHUNK = 128
NCHUNK = B_PER_W // CHUNK

NB_B = 4
NB = NB_B * L
GRID = N // NB

PAD_ROWS = 2000
PAD_GRID = V_WORD // PAD_ROWS


def _pad_body(t_ref, o_ref):
    o_ref[...] = jnp.concatenate(
        [t_ref[...], jnp.zeros((PAD_ROWS, D_PAD - D_WORD), jnp.float32)],
        axis=1)


def _pad_table(table):
    return pl.pallas_call(
        _pad_body,
        grid=(PAD_GRID,),
        in_specs=[pl.BlockSpec((PAD_ROWS, D_WORD), lambda i: (i, 0))],
        out_specs=pl.BlockSpec((PAD_ROWS, D_PAD), lambda i: (i, 0)),
        out_shape=jax.ShapeDtypeStruct((V_WORD, D_PAD), jnp.float32),
    )(table)


def _word_gather(idx_flat, table):
    @functools.partial(
        pl.kernel,
        out_type=jax.ShapeDtypeStruct((N, D_PAD), jnp.float32),
        mesh=plsc.VectorSubcoreMesh(core_axis_name="c", subcore_axis_name="s"),
        compiler_params=pltpu.CompilerParams(use_tc_tiling_on_sc=True),
        scratch_types=[
            pltpu.VMEM((CHUNK,), jnp.int32),
            pltpu.VMEM((CHUNK, D_PAD), jnp.float32),
            pltpu.SemaphoreType.DMA,
        ],
    )
    def k(idx_hbm, table_hbm, out_hbm, idx_v, rows_v, sem):
        wid = lax.axis_index("s") * NC + lax.axis_index("c")
        base = wid * B_PER_W

        def body(i, carry):
            off = pl.multiple_of(base + i * CHUNK, CHUNK)
            pltpu.sync_copy(idx_hbm.at[pl.ds(off, CHUNK)], idx_v)
            pltpu.async_copy(table_hbm.at[idx_v], rows_v, sem).wait()
            pltpu.sync_copy(rows_v, out_hbm.at[pl.ds(off, CHUNK)])
            return carry

        lax.fori_loop(0, NCHUNK, body, 0)

    return k(idx_flat, table)


def _dot_t(x, w):
    return lax.dot_general(x, w, (((1,), (1,)), ((), ())),
                           preferred_element_type=jnp.float32)


def _tc_body(w_ref, c_ref, ct_ref, wg0_ref, wt0_ref, wg1_ref, wt1_ref,
             b_ref, o_ref):
    wrows = w_ref[:, :D_WORD]
    chars = c_ref[...].reshape(NB, W)
    ctab = ct_ref[...]

    ids = lax.broadcasted_iota(jnp.int32, (NB, V_CHAR), 1)
    ce = None
    for j in range(W):
        oh = (chars[:, j][:, None] == ids).astype(jnp.bfloat16)
        e = jnp.dot(oh, ctab, preferred_element_type=jnp.float32)
        ce = e if ce is None else jnp.maximum(ce, e)

    x = jnp.concatenate([wrows, ce], axis=1)
    layers = ((wg0_ref, 0, wt0_ref, 1), (wg1_ref, 2, wt1_ref, 3))
    for wg_ref, bg_row, wt_ref, bt_row in layers:
        xb = x.astype(jnp.bfloat16)
        g = jax.nn.sigmoid(_dot_t(xb, wg_ref[...]) + b_ref[bg_row, :][None, :])
        t = jnp.maximum(_dot_t(xb, wt_ref[...]) + b_ref[bt_row, :][None, :],
                        0.0)
        x = g * t + (1.0 - g) * x
    o_ref[...] = x


def kernel(word_inputs, char_inputs, word_table, char_table,
           wt0, bt0, wg0, bg0, wt1, bt1, wg1, bg1):
    idx_flat = word_inputs.reshape(N)

    wrows = _word_gather(idx_flat, _pad_table(word_table))

    biases = jnp.concatenate(
        [jnp.stack([bg0, bt0, bg1, bt1]), jnp.zeros((4, HIDDEN), jnp.float32)],
        axis=0)

    rep = lambda i: (0, 0)
    out = pl.pallas_call(
        _tc_body,
        grid=(GRID,),
        in_specs=[
            pl.BlockSpec((NB, D_PAD), lambda i: (i, 0)),
            pl.BlockSpec((NB_B, L, W), lambda i: (i, 0, 0)),
            pl.BlockSpec((V_CHAR, D_CHAR), rep),
            pl.BlockSpec((HIDDEN, HIDDEN), rep),
            pl.BlockSpec((HIDDEN, HIDDEN), rep),
            pl.BlockSpec((HIDDEN, HIDDEN), rep),
            pl.BlockSpec((HIDDEN, HIDDEN), rep),
            pl.BlockSpec((8, HIDDEN), rep),
        ],
        out_specs=pl.BlockSpec((NB, HIDDEN), lambda i: (i, 0)),
        out_shape=jax.ShapeDtypeStruct((N, HIDDEN), jnp.float32),
    )(wrows, char_inputs, char_table.astype(jnp.bfloat16),
      wg0.astype(jnp.bfloat16), wt0.astype(jnp.bfloat16),
      wg1.astype(jnp.bfloat16), wt1.astype(jnp.bfloat16), biases)

    return out.reshape(B, L, HIDDEN)


# speedup vs baseline: 1.0636x; 1.0636x over previous
import functools

import jax
import jax.numpy as jnp
from jax import lax
from jax.experimental import pallas as pl
from jax.experimental.pallas import tpu as pltpu
from jax.experimental.pallas import tpu_sc as plsc

B = 1024
L = 200
N = B * L
D_WORD = 300
D_PAD = 384
D_CHAR = 64
V_WORD = 100000
V_CHAR = 128
W = 16
HIDDEN = D_WORD + D_CHAR

NC = 2
NS = 16
NW = NC * NS
B_PER_W = N // NW
CHUNK = 128
NCHUNK = B_PER_W // CHUNK

NB = 3200
GRID = N // NB

PAD_ROWS = 2000
PAD_GRID = V_WORD // PAD_ROWS


def _pad_body(t_ref, o_ref):
    o_ref[...] = jnp.concatenate(
        [t_ref[...], jnp.zeros((PAD_ROWS, D_PAD - D_WORD), jnp.float32)],
        axis=1)


def _pad_table(table):
    return pl.pallas_call(
        _pad_body,
        grid=(PAD_GRID,),
        in_specs=[pl.BlockSpec((PAD_ROWS, D_WORD), lambda i: (i, 0))],
        out_specs=pl.BlockSpec((PAD_ROWS, D_PAD), lambda i: (i, 0)),
        out_shape=jax.ShapeDtypeStruct((V_WORD, D_PAD), jnp.float32),
    )(table)


def _word_gather(idx_flat, table):
    @functools.partial(
        pl.kernel,
        out_type=jax.ShapeDtypeStruct((N, D_PAD), jnp.float32),
        mesh=plsc.VectorSubcoreMesh(core_axis_name="c", subcore_axis_name="s"),
        compiler_params=pltpu.CompilerParams(use_tc_tiling_on_sc=True),
        scratch_types=[
            pltpu.VMEM((CHUNK,), jnp.int32),
            pltpu.VMEM((CHUNK, D_PAD), jnp.float32),
            pltpu.SemaphoreType.DMA,
        ],
    )
    def k(idx_hbm, table_hbm, out_hbm, idx_v, rows_v, sem):
        wid = lax.axis_index("s") * NC + lax.axis_index("c")
        base = wid * B_PER_W

        def body(i, carry):
            off = pl.multiple_of(base + i * CHUNK, CHUNK)
            pltpu.sync_copy(idx_hbm.at[pl.ds(off, CHUNK)], idx_v)
            pltpu.async_copy(table_hbm.at[idx_v], rows_v, sem).wait()
            pltpu.sync_copy(rows_v, out_hbm.at[pl.ds(off, CHUNK)])
            return carry

        lax.fori_loop(0, NCHUNK, body, 0)

    return k(idx_flat, table)


def _dot_t(x, w):
    return lax.dot_general(x, w, (((1,), (1,)), ((), ())),
                           preferred_element_type=jnp.float32)


def _tc_body(w_ref, c_ref, ct_ref, w0_ref, w1_ref, b_ref, o_ref):
    wrows = w_ref[:, :D_WORD]
    ctab_t = ct_ref[...]

    ce_t = None
    for j in range(W):
        idxb = jnp.broadcast_to(c_ref[j, :][None, :], (D_CHAR, NB))
        e = jnp.take_along_axis(ctab_t, idxb, axis=1)
        ce_t = e if ce_t is None else jnp.maximum(ce_t, e)
    ce = ce_t.T

    x = jnp.concatenate([wrows, ce], axis=1)
    for w_cat_ref, bg_row, bt_row in ((w0_ref, 0, 1), (w1_ref, 2, 3)):
        z = jnp.dot(x.astype(jnp.bfloat16), w_cat_ref[...],
                    preferred_element_type=jnp.float32)
        g = jax.nn.sigmoid(z[:, :HIDDEN] + b_ref[bg_row, :][None, :])
        t = jnp.maximum(z[:, D_PAD:D_PAD + HIDDEN]
                        + b_ref[bt_row, :][None, :], 0.0)
        x = g * t + (1.0 - g) * x
    o_ref[...] = x


def kernel(word_inputs, char_inputs, word_table, char_table,
           wt0, bt0, wg0, bg0, wt1, bt1, wg1, bg1):
    idx_flat = word_inputs.reshape(N)

    wrows = _word_gather(idx_flat, _pad_table(word_table))

    biases = jnp.concatenate(
        [jnp.stack([bg0, bt0, bg1, bt1]), jnp.zeros((4, HIDDEN), jnp.float32)],
        axis=0)

    gap = jnp.zeros((HIDDEN, D_PAD - HIDDEN), jnp.bfloat16)
    tail = jnp.zeros((HIDDEN, 2 * D_PAD - D_PAD - HIDDEN), jnp.bfloat16)
    wcat0 = jnp.concatenate(
        [wg0.T.astype(jnp.bfloat16), gap, wt0.T.astype(jnp.bfloat16), tail],
        axis=1)
    wcat1 = jnp.concatenate(
        [wg1.T.astype(jnp.bfloat16), gap, wt1.T.astype(jnp.bfloat16), tail],
        axis=1)

    rep = lambda i: (0, 0)
    out = pl.pallas_call(
        _tc_body,
        grid=(GRID,),
        in_specs=[
            pl.BlockSpec((NB, D_PAD), lambda i: (i, 0)),
            pl.BlockSpec((W, NB), lambda i: (0, i)),
            pl.BlockSpec((D_CHAR, V_CHAR), rep),
            pl.BlockSpec((HIDDEN, 2 * D_PAD), rep),
            pl.BlockSpec((HIDDEN, 2 * D_PAD), rep),
            pl.BlockSpec((8, HIDDEN), rep),
        ],
        out_specs=pl.BlockSpec((NB, HIDDEN), lambda i: (i, 0)),
        out_shape=jax.ShapeDtypeStruct((N, HIDDEN), jnp.float32),
    )(wrows, char_inputs.reshape(N, W).T, char_table.T, wcat0, wcat1, biases)

    return out.reshape(B, L, HIDDEN)


# speedup vs baseline: 1.1557x; 1.0866x over previous
import functools

import jax
import jax.numpy as jnp
from jax import lax
from jax.experimental import pallas as pl
from jax.experimental.pallas import tpu as pltpu
from jax.experimental.pallas import tpu_sc as plsc

B = 1024
L = 200
N = B * L
D_WORD = 300
D_PAD = 384
D_CHAR = 64
V_WORD = 100000
V_CHAR = 128
W = 16
HIDDEN = D_WORD + D_CHAR

NC = 2
NS = 16
NW = NC * NS
QN = 4
NQ = N // QN
B_PER_W = NQ // NW
CHUNK = 64
NCHUNK = B_PER_W // CHUNK

NB = 3200
GRID = N // NB

PAD_ROWS = 2000
PAD_GRID = V_WORD // PAD_ROWS


def _pad_body(t_ref, o_ref):
    o_ref[...] = jnp.concatenate(
        [t_ref[...], jnp.zeros((PAD_ROWS, D_PAD - D_WORD), jnp.float32)],
        axis=1)


def _pad_table(table):
    return pl.pallas_call(
        _pad_body,
        grid=(PAD_GRID,),
        in_specs=[pl.BlockSpec((PAD_ROWS, D_WORD), lambda i: (i, 0))],
        out_specs=pl.BlockSpec((PAD_ROWS, D_PAD), lambda i: (i, 0)),
        out_shape=jax.ShapeDtypeStruct((V_WORD, D_PAD), jnp.float32),
    )(table)


def _word_gather(idx_flat, table):
    @functools.partial(
        pl.kernel,
        out_type=jax.ShapeDtypeStruct((NQ, D_PAD), jnp.float32),
        mesh=plsc.VectorSubcoreMesh(core_axis_name="c", subcore_axis_name="s"),
        compiler_params=pltpu.CompilerParams(use_tc_tiling_on_sc=True),
        scratch_types=[
            pltpu.VMEM((CHUNK,), jnp.int32),
            pltpu.VMEM((CHUNK, D_PAD), jnp.float32),
            pltpu.SemaphoreType.DMA,
        ],
    )
    def k(idx_hbm, table_hbm, out_hbm, idx_v, rows_v, sem):
        wid = lax.axis_index("s") * NC + lax.axis_index("c")
        base = wid * B_PER_W

        def body(i, carry):
            off = pl.multiple_of(base + i * CHUNK, CHUNK)
            pltpu.sync_copy(idx_hbm.at[pl.ds(off, CHUNK)], idx_v)
            pltpu.async_copy(table_hbm.at[idx_v], rows_v, sem).wait()
            pltpu.sync_copy(rows_v, out_hbm.at[pl.ds(off, CHUNK)])
            return carry

        lax.fori_loop(0, NCHUNK, body, 0)

    return k(idx_flat, table)


def _dot_t(x, w):
    return lax.dot_general(x, w, (((1,), (1,)), ((), ())),
                           preferred_element_type=jnp.float32)


def _tc_body(w_ref, c_ref, ct_ref, w0_ref, w1_ref, b_ref, o_ref):
    wrows = w_ref[:, :D_WORD]
    ctab_t = ct_ref[...]

    ce_t = None
    for j in range(W):
        idxb = jnp.broadcast_to(c_ref[j, :][None, :], (D_CHAR, NB))
        e = jnp.take_along_axis(ctab_t, idxb, axis=1)
        ce_t = e if ce_t is None else jnp.maximum(ce_t, e)
    ce = ce_t.T

    x = jnp.concatenate([wrows, ce], axis=1)
    for w_cat_ref, bg_row, bt_row in ((w0_ref, 0, 1), (w1_ref, 2, 3)):
        z = jnp.dot(x.astype(jnp.bfloat16), w_cat_ref[...],
                    preferred_element_type=jnp.float32)
        g = jax.nn.sigmoid(z[:, :HIDDEN] + b_ref[bg_row, :][None, :])
        t = jnp.maximum(z[:, D_PAD:D_PAD + HIDDEN]
                        + b_ref[bt_row, :][None, :], 0.0)
        x = g * t + (1.0 - g) * x
    o_ref[...] = x


def kernel(word_inputs, char_inputs, word_table, char_table,
           wt0, bt0, wg0, bg0, wt1, bt1, wg1, bg1):
    idx_flat = word_inputs.reshape(N)
    chars_t = char_inputs.reshape(N, W).T
    table_pad = _pad_table(word_table)

    biases = jnp.concatenate(
        [jnp.stack([bg0, bt0, bg1, bt1]), jnp.zeros((4, HIDDEN), jnp.float32)],
        axis=0)

    gap = jnp.zeros((HIDDEN, D_PAD - HIDDEN), jnp.bfloat16)
    tail = jnp.zeros((HIDDEN, 2 * D_PAD - D_PAD - HIDDEN), jnp.bfloat16)
    wcat0 = jnp.concatenate(
        [wg0.T.astype(jnp.bfloat16), gap, wt0.T.astype(jnp.bfloat16), tail],
        axis=1)
    wcat1 = jnp.concatenate(
        [wg1.T.astype(jnp.bfloat16), gap, wt1.T.astype(jnp.bfloat16), tail],
        axis=1)

    rep = lambda i: (0, 0)
    grid_q = NQ // NB
    outs = []
    for q in range(QN):
        wrows_q = _word_gather(
            lax.slice_in_dim(idx_flat, q * NQ, (q + 1) * NQ), table_pad)
        out_q = pl.pallas_call(
            _tc_body,
            grid=(grid_q,),
            in_specs=[
                pl.BlockSpec((NB, D_PAD), lambda i: (i, 0)),
                pl.BlockSpec((W, NB), lambda i, _q=q: (0, i + _q * grid_q)),
                pl.BlockSpec((D_CHAR, V_CHAR), rep),
                pl.BlockSpec((HIDDEN, 2 * D_PAD), rep),
                pl.BlockSpec((HIDDEN, 2 * D_PAD), rep),
                pl.BlockSpec((8, HIDDEN), rep),
            ],
            out_specs=pl.BlockSpec((NB, HIDDEN), lambda i: (i, 0)),
            out_shape=jax.ShapeDtypeStruct((NQ, HIDDEN), jnp.float32),
        )(wrows_q, chars_t, char_table.T, wcat0, wcat1, biases)
        outs.append(out_q.reshape(B // QN, L, HIDDEN))

    return jnp.concatenate(outs, axis=0)
